# baseline (speedup 1.0000x reference)
import functools

import jax
import jax.numpy as jnp
from jax import lax
from jax.experimental import pallas as pl
from jax.experimental.pallas import tpu as pltpu
from jax.experimental.pallas import tpu_sc as plsc

B, C, H, W = 8, 2, 512, 512
HW = H * W
NC, NS, L = 2, 16, 16
BPC = B // NC
ROWS_PER_TEC = H // NS
PX_PER_TEC = ROWS_PER_TEC * W
CH = 1024
NCHUNK = PX_PER_TEC // CH
NB = 1024
NG = PX_PER_TEC // NB

_mesh = plsc.VectorSubcoreMesh(core_axis_name="c", subcore_axis_name="s")

_scratch = []
for _ in range(2):
    _scratch += [
        pltpu.VMEM_SHARED((HW,), jnp.float32),
        pltpu.VMEM_SHARED((HW,), jnp.float32),
        pltpu.VMEM_SHARED((HW,), jnp.float32),
    ]
for _ in range(2):
    _scratch += [
        pltpu.VMEM((CH,), jnp.float32),
        pltpu.VMEM((CH,), jnp.float32),
    ]
for _ in range(2):
    _scratch += [
        pltpu.VMEM((CH,), jnp.float32),
        pltpu.VMEM((CH,), jnp.float32),
        pltpu.VMEM((CH,), jnp.float32),
        pltpu.VMEM((CH,), jnp.int32),
        pltpu.VMEM((CH,), jnp.int32),
        pltpu.VMEM((CH,), jnp.int32),
        pltpu.VMEM((CH,), jnp.int32),
    ]
_scratch += [pltpu.VMEM((NB,), jnp.float32)]
for _ in range(2):
    _scratch += [
        pltpu.VMEM((NB,), jnp.float32),
        pltpu.VMEM((NB,), jnp.float32),
        pltpu.VMEM((NB,), jnp.float32),
        pltpu.VMEM((NB,), jnp.float32),
        pltpu.VMEM((NB,), jnp.float32),
    ]
_scratch += [pltpu.SemaphoreType.DMA] * 8


@functools.partial(
    pl.kernel,
    mesh=_mesh,
    out_type=jax.ShapeDtypeStruct((B, C, HW), jnp.float32),
    scratch_types=_scratch,
)
def _warp_kernel(in_hbm, out_hbm, *rest):
    accs = [rest[3 * q:3 * q + 3] for q in range(2)]
    inbufs = [rest[6:8], rest[8:10]]
    cbufs = [rest[10 + 7 * p:10 + 7 * p + 7] for p in range(2)]
    zbuf = rest[24]
    nbufs = [rest[25 + 5 * p:25 + 5 * p + 5] for p in range(2)]
    sem_stream = rest[35]
    sem_load = rest[36:38]
    sem_store = rest[38:40]
    sem_zero = rest[40]
    sem_in = rest[41:43]
    cid = lax.axis_index("c")
    sid = lax.axis_index("s")
    px0 = sid * PX_PER_TEC
    lanes = lax.iota(jnp.int32, L)
    zeros16 = jnp.zeros((L,), jnp.float32)
    lim = jnp.float32(W - 1)

    def zfill(k, _):
        zbuf[pl.ds(k * L, L)] = zeros16
        return 0
    lax.fori_loop(0, NB // L, zfill, 0)

    def fire_zero(q, g):
        base = px0 + g * NB
        return [pltpu.async_copy(zbuf, acc.at[pl.ds(base, NB)], sem_zero)
                for acc in accs[q]]

    def wait_zero(q, g):
        base = px0 + g * NB
        for acc in accs[q]:
            pltpu.make_async_copy(zbuf, acc.at[pl.ds(base, NB)],
                                  sem_zero).wait()

    def in_copies(p, b, ci):
        fxb, fyb = inbufs[p]
        cbase = px0 + ci * CH
        return [
            (in_hbm.at[b, 0, pl.ds(cbase, CH)], fxb),
            (in_hbm.at[b, 1, pl.ds(cbase, CH)], fyb),
        ]

    def fire_input(p, b, ci):
        for src, dst in in_copies(p, b, ci):
            pltpu.async_copy(src, dst, sem_in[p])

    def wait_input(p, b, ci):
        for src, dst in in_copies(p, b, ci):
            pltpu.make_async_copy(src, dst, sem_in[p]).wait()

    def compute_chunk(p, ci):
        fxb, fyb = inbufs[p]
        vxb, vyb, wb, idx0, idx1, idx2, idx3 = cbufs[p]
        cbase = px0 + ci * CH

        def vec(kv, _):
            k = kv * L
            p_ = cbase + k + lanes
            row = lax.shift_right_logical(p_, 9)
            col = lax.bitwise_and(p_, W - 1)
            fxv = fxb[pl.ds(k, L)]
            fyv = fyb[pl.ds(k, L)]
            x2 = col.astype(jnp.float32) + fxv
            y2 = row.astype(jnp.float32) + fyv
            valid = ((x2 >= 0.0) & (y2 >= 0.0)
                     & (x2 <= lim) & (y2 <= lim))
            w1 = jnp.where(valid, jnp.float32(1.0), jnp.float32(0.0))
            ixl = jnp.clip(x2.astype(jnp.int32), 0, W - 1)
            iyt = jnp.clip(y2.astype(jnp.int32), 0, H - 1)
            ixr = jnp.minimum(ixl + 1, W - 1)
            iyb = jnp.minimum(iyt + 1, H - 1)
            vxb[pl.ds(k, L)] = -fxv * w1
            vyb[pl.ds(k, L)] = -fyv * w1
            wb[pl.ds(k, L)] = w1
            rt = iyt * W
            rb = iyb * W
            idx0[pl.ds(k, L)] = rt + ixl
            idx1[pl.ds(k, L)] = rt + ixr
            idx2[pl.ds(k, L)] = rb + ixl
            idx3[pl.ds(k, L)] = rb + ixr
            return 0
        lax.fori_loop(0, CH // L, vec, 0)

    def fire_chunk(q, p):
        accx, accy, accc = accs[q]
        vxb, vyb, wb, idx0, idx1, idx2, idx3 = cbufs[p]
        copies = []
        for ixr_ref in (idx0, idx1, idx2, idx3):
            copies.append(pltpu.async_copy(
                vxb, accx.at[ixr_ref], sem_stream, add=True))
            copies.append(pltpu.async_copy(
                vyb, accy.at[ixr_ref], sem_stream, add=True))
            copies.append(pltpu.async_copy(
                wb, accc.at[ixr_ref], sem_stream, add=True))
        return copies

    def fire_load(q, g, p):
        base = px0 + g * NB
        nbx, nby, nbc, _, _ = nbufs[p]
        accx, accy, accc = accs[q]
        for src, dst in ((accx, nbx), (accy, nby), (accc, nbc)):
            pltpu.async_copy(src.at[pl.ds(base, NB)], dst, sem_load[p])

    def wait_load(q, g, p):
        base = px0 + g * NB
        nbx, nby, nbc, _, _ = nbufs[p]
        accx, accy, accc = accs[q]
        for src, dst in ((accx, nbx), (accy, nby), (accc, nbc)):
            pltpu.make_async_copy(src.at[pl.ds(base, NB)], dst,
                                  sem_load[p]).wait()

    def store_copies(bn, g, p):
        base = px0 + g * NB
        _, _, _, oxb, oyb = nbufs[p]
        return [
            (oxb, out_hbm.at[bn, 0, pl.ds(base, NB)]),
            (oyb, out_hbm.at[bn, 1, pl.ds(base, NB)]),
        ]

    def fire_store(bn, g, p):
        for src, dst in store_copies(bn, g, p):
            pltpu.async_copy(src, dst, sem_store[p])

    def wait_store(bn, g, p):
        for src, dst in store_copies(bn, g, p):
            pltpu.make_async_copy(src, dst, sem_store[p]).wait()

    def norm_group(qn, bn, t, p):
        wait_load(qn, t, p)
        fire_zero(qn, t)
        wait_store(bn, t, p)
        nbx, nby, nbc, oxb, oyb = nbufs[p]

        def nv(kv, _):
            k = kv * L
            sx = nbx[pl.ds(k, L)]
            sy = nby[pl.ds(k, L)]
            cc = nbc[pl.ds(k, L)]
            r = jnp.float32(1.0) / jnp.where(cc > 0.0, cc,
                                             jnp.float32(1.0))
            oxb[pl.ds(k, L)] = sx * r
            oyb[pl.ds(k, L)] = sy * r
            return 0
        lax.fori_loop(0, NB // L, nv, 0)
        fire_store(bn, t, p)

    for g in range(NG):
        fire_zero(0, g)
    for g in range(NG):
        wait_zero(0, g)
    plsc.subcore_barrier()

    for s in range(BPC + 1):
        q = s % 2
        do_scatter = s < BPC
        do_norm = s > 0
        b = cid * BPC + s
        bn = cid * BPC + (s - 1)

        if do_scatter:
            fire_input(0, b, 0)
            fire_input(1, b, 1)
        if do_norm:
            fire_store(bn, 0, 0)
            fire_store(bn, 1, 1)

        def body(g, _):
            t0 = 2 * g
            t1 = 2 * g + 1
            if do_norm:
                fire_load(1 - q, t0, 0)
                fire_load(1 - q, t1, 1)
            copies_a = None
            if do_scatter:
                wait_input(0, b, t0)
                compute_chunk(0, t0)
                fire_input(0, b, jnp.minimum(t0 + 2, NCHUNK - 2))
                copies_a = fire_chunk(q, 0)
            if do_norm:
                norm_group(1 - q, bn, t0, 0)
            if do_scatter:
                wait_input(1, b, t1)
                compute_chunk(1, t1)
                fire_input(1, b, jnp.minimum(t1 + 2, NCHUNK - 1))
                for cp in copies_a:
                    cp.wait()
                copies_b = fire_chunk(q, 1)
            if do_norm:
                norm_group(1 - q, bn, t1, 1)
            if do_scatter:
                for cp in copies_b:
                    cp.wait()
            return 0
        lax.fori_loop(0, NCHUNK // 2, body, 0)

        if do_scatter:
            wait_input(0, b, NCHUNK - 2)
            wait_input(1, b, NCHUNK - 1)
        if do_norm:
            wait_store(bn, NG - 2, 0)
            wait_store(bn, NG - 1, 1)
            for g in range(NG):
                wait_zero(1 - q, g)
        plsc.subcore_barrier()


def kernel(input1):
    flat = input1.reshape(B, C, HW)
    out = _warp_kernel(flat)
    return out.reshape(B, C, H, W)


# speedup vs baseline: 1.0753x; 1.0753x over previous
import functools

import jax
import jax.numpy as jnp
from jax import lax
from jax.experimental import pallas as pl
from jax.experimental.pallas import tpu as pltpu
from jax.experimental.pallas import tpu_sc as plsc

B, C, H, W = 8, 2, 512, 512
HW = H * W
NC, NS, L = 2, 16, 16
BPC = B // NC
ROWS_PER_TEC = H // NS
PX_PER_TEC = ROWS_PER_TEC * W
CH = 2048
NCHUNK = PX_PER_TEC // CH
NB = 2048
NG = PX_PER_TEC // NB

_mesh = plsc.VectorSubcoreMesh(core_axis_name="c", subcore_axis_name="s")

_scratch = [
    pltpu.VMEM_SHARED((HW,), jnp.float32),
    pltpu.VMEM_SHARED((HW,), jnp.float32),
    pltpu.VMEM_SHARED((HW,), jnp.float32),
    pltpu.VMEM((CH,), jnp.float32),
    pltpu.VMEM((CH,), jnp.float32),
    pltpu.VMEM((CH,), jnp.float32),
    pltpu.VMEM((CH,), jnp.float32),
]
for _ in range(2):
    _scratch += [
        pltpu.VMEM((CH,), jnp.float32),
        pltpu.VMEM((CH,), jnp.float32),
        pltpu.VMEM((CH,), jnp.float32),
        pltpu.VMEM((CH,), jnp.int32),
        pltpu.VMEM((CH,), jnp.int32),
        pltpu.VMEM((CH,), jnp.int32),
        pltpu.VMEM((CH,), jnp.int32),
    ]
_scratch += [pltpu.VMEM((NB,), jnp.float32)]
for _ in range(2):
    _scratch += [
        pltpu.VMEM((NB,), jnp.float32),
        pltpu.VMEM((NB,), jnp.float32),
        pltpu.VMEM((NB,), jnp.float32),
        pltpu.VMEM((NB,), jnp.float32),
        pltpu.VMEM((NB,), jnp.float32),
    ]
_scratch += [pltpu.SemaphoreType.DMA] * 8


@functools.partial(
    pl.kernel,
    mesh=_mesh,
    out_type=jax.ShapeDtypeStruct((B, C, HW), jnp.float32),
    scratch_types=_scratch,
)
def _warp_kernel(in_hbm, out_hbm, accx, accy, accc, *rest):
    inbufs = [rest[0:2], rest[2:4]]
    cbufs = [rest[4 + 7 * p:4 + 7 * p + 7] for p in range(2)]
    zbuf = rest[18]
    nbufs = [rest[19 + 5 * p:19 + 5 * p + 5] for p in range(2)]
    sem_stream = rest[29]
    sem_load = rest[30:32]
    sem_store = rest[32:34]
    sem_zero = rest[34]
    sem_in = rest[35:37]
    cid = lax.axis_index("c")
    sid = lax.axis_index("s")
    px0 = sid * PX_PER_TEC
    lanes = lax.iota(jnp.int32, L)
    zeros16 = jnp.zeros((L,), jnp.float32)
    lim = jnp.float32(W - 1)

    def zfill(k, _):
        zbuf[pl.ds(k * L, L)] = zeros16
        return 0
    lax.fori_loop(0, NB // L, zfill, 0)

    def fire_zero(g):
        base = px0 + g * NB
        return [
            pltpu.async_copy(zbuf, accx.at[pl.ds(base, NB)], sem_zero),
            pltpu.async_copy(zbuf, accy.at[pl.ds(base, NB)], sem_zero),
            pltpu.async_copy(zbuf, accc.at[pl.ds(base, NB)], sem_zero),
        ]

    def fire_input(p, b, ci):
        fxb, fyb = inbufs[p]
        cbase = px0 + ci * CH
        return [
            pltpu.async_copy(in_hbm.at[b, 0, pl.ds(cbase, CH)], fxb,
                             sem_in[p]),
            pltpu.async_copy(in_hbm.at[b, 1, pl.ds(cbase, CH)], fyb,
                             sem_in[p]),
        ]

    def compute_chunk(p, ci):
        fxb, fyb = inbufs[p]
        vxb, vyb, wb, idx0, idx1, idx2, idx3 = cbufs[p]
        cbase = px0 + ci * CH

        def vec_body(k):
            p_ = cbase + k + lanes
            row = lax.shift_right_logical(p_, 9)
            col = lax.bitwise_and(p_, W - 1)
            fxv = fxb[pl.ds(k, L)]
            fyv = fyb[pl.ds(k, L)]
            x2 = col.astype(jnp.float32) + fxv
            y2 = row.astype(jnp.float32) + fyv
            valid = ((x2 >= 0.0) & (y2 >= 0.0)
                     & (x2 <= lim) & (y2 <= lim))
            w1 = jnp.where(valid, jnp.float32(1.0), jnp.float32(0.0))
            ixl = jnp.clip(x2.astype(jnp.int32), 0, W - 1)
            iyt = jnp.clip(y2.astype(jnp.int32), 0, H - 1)
            ixr = jnp.minimum(ixl + 1, W - 1)
            iyb = jnp.minimum(iyt + 1, H - 1)
            vxb[pl.ds(k, L)] = -fxv * w1
            vyb[pl.ds(k, L)] = -fyv * w1
            wb[pl.ds(k, L)] = w1
            rt = iyt * W
            rb = iyb * W
            idx0[pl.ds(k, L)] = rt + ixl
            idx1[pl.ds(k, L)] = rt + ixr
            idx2[pl.ds(k, L)] = rb + ixl
            idx3[pl.ds(k, L)] = rb + ixr

        def vec(kv, _):
            vec_body(kv * (2 * L))
            vec_body(kv * (2 * L) + L)
            return 0
        lax.fori_loop(0, CH // (2 * L), vec, 0)

    def fire_chunk(p):
        vxb, vyb, wb, idx0, idx1, idx2, idx3 = cbufs[p]
        copies = []
        for ixr_ref in (idx0, idx1, idx2, idx3):
            copies.append(pltpu.async_copy(
                vxb, accx.at[ixr_ref], sem_stream, add=True))
            copies.append(pltpu.async_copy(
                vyb, accy.at[ixr_ref], sem_stream, add=True))
            copies.append(pltpu.async_copy(
                wb, accc.at[ixr_ref], sem_stream, add=True))
        return copies

    def fire_load(g, p):
        base = px0 + g * NB
        nbx, nby, nbc, _, _ = nbufs[p]
        return [
            pltpu.async_copy(accx.at[pl.ds(base, NB)], nbx, sem_load[p]),
            pltpu.async_copy(accy.at[pl.ds(base, NB)], nby, sem_load[p]),
            pltpu.async_copy(accc.at[pl.ds(base, NB)], nbc, sem_load[p]),
        ]

    def do_batch(b):
        inflight = None
        in_flt = [None, None]
        in_flt[0] = fire_input(0, b, 0)
        for ci in range(NCHUNK):
            p = ci % 2
            if ci + 1 < NCHUNK:
                in_flt[1 - p] = fire_input(1 - p, b, ci + 1)
            for cp in in_flt[p]:
                cp.wait()
            compute_chunk(p, ci)
            if inflight is not None:
                for cp in inflight:
                    cp.wait()
            inflight = fire_chunk(p)
        for cp in inflight:
            cp.wait()
        plsc.subcore_barrier()

        zero_copies = []
        store_copies = [None, None]
        loads = [None, None]
        loads[0] = fire_load(0, 0)
        for g in range(NG):
            p = g % 2
            if g + 1 < NG:
                loads[1 - p] = fire_load(g + 1, 1 - p)
            for cp in loads[p]:
                cp.wait()
            zero_copies += fire_zero(g)
            nbx, nby, nbc, oxb, oyb = nbufs[p]
            if store_copies[p] is not None:
                for cp in store_copies[p]:
                    cp.wait()

            def nv_body(k):
                sx = nbx[pl.ds(k, L)]
                sy = nby[pl.ds(k, L)]
                cc = nbc[pl.ds(k, L)]
                r = jnp.float32(1.0) / jnp.where(cc > 0.0, cc,
                                                 jnp.float32(1.0))
                oxb[pl.ds(k, L)] = sx * r
                oyb[pl.ds(k, L)] = sy * r

            def nv(kv, _):
                nv_body(kv * (2 * L))
                nv_body(kv * (2 * L) + L)
                return 0
            lax.fori_loop(0, NB // (2 * L), nv, 0)
            base = px0 + g * NB
            store_copies[p] = [
                pltpu.async_copy(oxb, out_hbm.at[b, 0, pl.ds(base, NB)],
                                 sem_store[p]),
                pltpu.async_copy(oyb, out_hbm.at[b, 1, pl.ds(base, NB)],
                                 sem_store[p]),
            ]
        for sc_list in store_copies:
            if sc_list is not None:
                for cp in sc_list:
                    cp.wait()
        for cp in zero_copies:
            cp.wait()
        plsc.subcore_barrier()

    prol = []
    for g in range(NG):
        prol += fire_zero(g)
    for cp in prol:
        cp.wait()
    plsc.subcore_barrier()

    def batch_loop(bi, _):
        do_batch(cid * BPC + bi)
        return 0
    lax.fori_loop(0, BPC, batch_loop, 0)


def kernel(input1):
    flat = input1.reshape(B, C, HW)
    out = _warp_kernel(flat)
    return out.reshape(B, C, H, W)
